# baseline (device time: 10452 ns/iter reference)
import jax
import jax.numpy as jnp
from jax import lax
from jax.experimental import pallas as pl
from jax.experimental.pallas import tpu as pltpu

N_BCHUNK = 4
N_ACHUNK = 2


def kernel(A, B):
    m, k = A.shape
    _, n = B.shape
    nh = n // 2
    kb = k // N_BCHUNK
    ka = k // N_ACHUNK

    def body(a_hbm, b_hbm, out_hbm, a_ref, b_ref, bmy_ref, a16_ref,
             bhalf_ref, a_rx, bdir_rx, bfwd_rx, out_vmem, load_sems,
             store_sems, send_sems, a_recv_sems, bdir_sems, bfwd_sems):
        my_x = lax.axis_index("x")
        my_y = lax.axis_index("y")
        peer = (my_x, 1 - my_y)
        xnbr = (1 - my_x, my_y)

        barrier_sem = pltpu.get_barrier_semaphore()
        for nbr in (peer, xnbr):
            pl.semaphore_signal(
                barrier_sem, inc=1, device_id=nbr,
                device_id_type=pl.DeviceIdType.MESH,
            )

        bmy_cp = pltpu.make_async_copy(
            b_hbm.at[:, pl.ds(my_x * nh, nh)], bmy_ref, load_sems.at[2]
        )
        bmy_cp.start()
        b_cp = pltpu.make_async_copy(b_hbm, b_ref, load_sems.at[1])
        a_cp = pltpu.make_async_copy(a_hbm, a_ref, load_sems.at[0])
        b_cp.start()
        a_cp.start()

        bmy_cp.wait()
        bhalf_ref[...] = bmy_ref[...].astype(jnp.bfloat16)

        pl.semaphore_wait(barrier_sem, 2)

        sends = []
        for c in range(N_BCHUNK):
            rows = pl.ds(c * kb, kb)
            s = pltpu.make_async_remote_copy(
                src_ref=bhalf_ref.at[rows, :],
                dst_ref=bdir_rx.at[rows, :],
                send_sem=send_sems.at[c],
                recv_sem=bdir_sems.at[c],
                device_id=peer,
                device_id_type=pl.DeviceIdType.MESH,
            )
            s.start()
            sends.append(s)

        a_cp.wait()
        a16_ref[...] = a_ref[...].astype(jnp.bfloat16)
        b_cp.wait()
        b16 = b_ref[...].astype(jnp.bfloat16)
        a_sends = []
        for c in range(N_ACHUNK):
            cols = pl.ds(c * ka, ka)
            s = pltpu.make_async_remote_copy(
                src_ref=a16_ref.at[:, cols],
                dst_ref=a_rx.at[:, cols],
                send_sem=send_sems.at[N_BCHUNK + c],
                recv_sem=a_recv_sems.at[c],
                device_id=peer,
                device_id_type=pl.DeviceIdType.MESH,
            )
            s.start()
            a_sends.append(s)

        fwds = []
        for c in range(N_BCHUNK):
            rows = pl.ds(c * kb, kb)
            sends[c].wait_recv()
            f = pltpu.make_async_remote_copy(
                src_ref=bdir_rx.at[rows, :],
                dst_ref=bfwd_rx.at[rows, :],
                send_sem=send_sems.at[N_BCHUNK + N_ACHUNK + c],
                recv_sem=bfwd_sems.at[c],
                device_id=xnbr,
                device_id_type=pl.DeviceIdType.MESH,
            )
            f.start()
            fwds.append(f)

        acc_l = jnp.dot(a16_ref[...], b16[:, :nh],
                        preferred_element_type=jnp.float32)
        acc_r = jnp.dot(a16_ref[...], b16[:, nh:],
                        preferred_element_type=jnp.float32)

        rel = ka // kb
        for c in range(N_ACHUNK):
            a_sends[c].wait_recv()
            for r in range(c * rel, (c + 1) * rel):
                fwds[r].wait_recv()
            a_c = a_rx[:, c * ka:(c + 1) * ka]
            bdir_c = bdir_rx[c * ka:(c + 1) * ka, :]
            bfwd_c = bfwd_rx[c * ka:(c + 1) * ka, :]
            bl_c = jnp.where(my_x == 0, bdir_c, bfwd_c)
            br_c = jnp.where(my_x == 0, bfwd_c, bdir_c)
            acc_l = acc_l + jnp.dot(a_c, bl_c,
                                    preferred_element_type=jnp.float32)
            acc_r = acc_r + jnp.dot(a_c, br_c,
                                    preferred_element_type=jnp.float32)

        out_vmem[:, :nh] = acc_l.astype(jnp.bfloat16)
        st_l = pltpu.make_async_copy(
            out_vmem.at[:, :nh], out_hbm.at[:, :nh], store_sems.at[0]
        )
        st_l.start()
        out_vmem[:, nh:] = acc_r.astype(jnp.bfloat16)
        st_r = pltpu.make_async_copy(
            out_vmem.at[:, nh:], out_hbm.at[:, nh:], store_sems.at[1]
        )
        st_r.start()
        st_l.wait()
        st_r.wait()

        for s in sends + a_sends + fwds:
            s.wait_send()

    return pl.pallas_call(
        body,
        out_shape=jax.ShapeDtypeStruct((m, n), jnp.bfloat16),
        in_specs=[
            pl.BlockSpec(memory_space=pltpu.MemorySpace.HBM),
            pl.BlockSpec(memory_space=pltpu.MemorySpace.HBM),
        ],
        out_specs=pl.BlockSpec(memory_space=pltpu.MemorySpace.HBM),
        scratch_shapes=[
            pltpu.VMEM((m, k), jnp.float32),
            pltpu.VMEM((k, n), jnp.float32),
            pltpu.VMEM((k, n // 2), jnp.float32),
            pltpu.VMEM((m, k), jnp.bfloat16),
            pltpu.VMEM((k, n // 2), jnp.bfloat16),
            pltpu.VMEM((m, k), jnp.bfloat16),
            pltpu.VMEM((k, n // 2), jnp.bfloat16),
            pltpu.VMEM((k, n // 2), jnp.bfloat16),
            pltpu.VMEM((m, n), jnp.bfloat16),
            pltpu.SemaphoreType.DMA((3,)),
            pltpu.SemaphoreType.DMA((2,)),
            pltpu.SemaphoreType.DMA((2 * N_BCHUNK + N_ACHUNK,)),
            pltpu.SemaphoreType.DMA((N_ACHUNK,)),
            pltpu.SemaphoreType.DMA((N_BCHUNK,)),
            pltpu.SemaphoreType.DMA((N_BCHUNK,)),
        ],
        compiler_params=pltpu.CompilerParams(collective_id=0),
    )(
        pltpu.with_memory_space_constraint(A, pltpu.MemorySpace.HBM),
        pltpu.with_memory_space_constraint(B, pltpu.MemorySpace.HBM),
    )


# device time: 10375 ns/iter; 1.0074x vs baseline; 1.0074x over previous
import jax
import jax.numpy as jnp
from jax import lax
from jax.experimental import pallas as pl
from jax.experimental.pallas import tpu as pltpu

N_BCHUNK = 4
N_ACHUNK = 2


def kernel(A, B):
    m, k = A.shape
    _, n = B.shape
    nh = n // 2
    kb = k // N_BCHUNK
    ka = k // N_ACHUNK

    def body(a_hbm, b_hbm, out_ref, a_ref, b_ref, a16_ref, bhalf_ref, a_rx,
             bdir_rx, bfwd_rx, load_sems, send_sems, a_recv_sems, bdir_sems,
             bfwd_sems):
        my_x = lax.axis_index("x")
        my_y = lax.axis_index("y")
        peer = (my_x, 1 - my_y)
        xnbr = (1 - my_x, my_y)

        barrier_sem = pltpu.get_barrier_semaphore()
        for nbr in (peer, xnbr):
            pl.semaphore_signal(
                barrier_sem, inc=1, device_id=nbr,
                device_id_type=pl.DeviceIdType.MESH,
            )

        b_cp = pltpu.make_async_copy(b_hbm, b_ref, load_sems.at[1])
        a_cp = pltpu.make_async_copy(a_hbm, a_ref, load_sems.at[0])
        b_cp.start()
        a_cp.start()

        b_cp.wait()
        b16 = b_ref[...].astype(jnp.bfloat16)
        bhalf_ref[...] = jnp.where(my_x == 0, b16[:, :nh], b16[:, nh:])

        pl.semaphore_wait(barrier_sem, 2)

        sends = []
        for c in range(N_BCHUNK):
            rows = pl.ds(c * kb, kb)
            s = pltpu.make_async_remote_copy(
                src_ref=bhalf_ref.at[rows, :],
                dst_ref=bdir_rx.at[rows, :],
                send_sem=send_sems.at[c],
                recv_sem=bdir_sems.at[c],
                device_id=peer,
                device_id_type=pl.DeviceIdType.MESH,
            )
            s.start()
            sends.append(s)

        a_cp.wait()
        a16_ref[...] = a_ref[...].astype(jnp.bfloat16)
        a_sends = []
        for c in range(N_ACHUNK):
            cols = pl.ds(c * ka, ka)
            s = pltpu.make_async_remote_copy(
                src_ref=a16_ref.at[:, cols],
                dst_ref=a_rx.at[:, cols],
                send_sem=send_sems.at[N_BCHUNK + c],
                recv_sem=a_recv_sems.at[c],
                device_id=peer,
                device_id_type=pl.DeviceIdType.MESH,
            )
            s.start()
            a_sends.append(s)

        fwds = []
        for c in range(N_BCHUNK):
            rows = pl.ds(c * kb, kb)
            sends[c].wait_recv()
            f = pltpu.make_async_remote_copy(
                src_ref=bdir_rx.at[rows, :],
                dst_ref=bfwd_rx.at[rows, :],
                send_sem=send_sems.at[N_BCHUNK + N_ACHUNK + c],
                recv_sem=bfwd_sems.at[c],
                device_id=xnbr,
                device_id_type=pl.DeviceIdType.MESH,
            )
            f.start()
            fwds.append(f)

        acc_l = jnp.dot(a16_ref[...], b16[:, :nh],
                        preferred_element_type=jnp.float32)
        acc_r = jnp.dot(a16_ref[...], b16[:, nh:],
                        preferred_element_type=jnp.float32)

        rel = ka // kb
        for c in range(N_ACHUNK):
            a_sends[c].wait_recv()
            for r in range(c * rel, (c + 1) * rel):
                fwds[r].wait_recv()
            a_c = a_rx[:, c * ka:(c + 1) * ka]
            bdir_c = bdir_rx[c * ka:(c + 1) * ka, :]
            bfwd_c = bfwd_rx[c * ka:(c + 1) * ka, :]
            bl_c = jnp.where(my_x == 0, bdir_c, bfwd_c)
            br_c = jnp.where(my_x == 0, bfwd_c, bdir_c)
            acc_l = acc_l + jnp.dot(a_c, bl_c,
                                    preferred_element_type=jnp.float32)
            acc_r = acc_r + jnp.dot(a_c, br_c,
                                    preferred_element_type=jnp.float32)

        out_ref[:, :nh] = acc_l.astype(jnp.bfloat16)
        out_ref[:, nh:] = acc_r.astype(jnp.bfloat16)

        for s in sends + a_sends + fwds:
            s.wait_send()

    return pl.pallas_call(
        body,
        out_shape=jax.ShapeDtypeStruct((m, n), jnp.bfloat16),
        in_specs=[
            pl.BlockSpec(memory_space=pltpu.MemorySpace.HBM),
            pl.BlockSpec(memory_space=pltpu.MemorySpace.HBM),
        ],
        out_specs=pl.BlockSpec(memory_space=pltpu.VMEM),
        scratch_shapes=[
            pltpu.VMEM((m, k), jnp.float32),
            pltpu.VMEM((k, n), jnp.float32),
            pltpu.VMEM((m, k), jnp.bfloat16),
            pltpu.VMEM((k, n // 2), jnp.bfloat16),
            pltpu.VMEM((m, k), jnp.bfloat16),
            pltpu.VMEM((k, n // 2), jnp.bfloat16),
            pltpu.VMEM((k, n // 2), jnp.bfloat16),
            pltpu.SemaphoreType.DMA((2,)),
            pltpu.SemaphoreType.DMA((2 * N_BCHUNK + N_ACHUNK,)),
            pltpu.SemaphoreType.DMA((N_ACHUNK,)),
            pltpu.SemaphoreType.DMA((N_BCHUNK,)),
            pltpu.SemaphoreType.DMA((N_BCHUNK,)),
        ],
        compiler_params=pltpu.CompilerParams(collective_id=0),
    )(
        pltpu.with_memory_space_constraint(A, pltpu.MemorySpace.HBM),
        pltpu.with_memory_space_constraint(B, pltpu.MemorySpace.HBM),
    )
